# baseline (speedup 1.0000x reference)
import functools

import jax
import jax.numpy as jnp
from jax import lax
from jax.experimental import pallas as pl
from jax.experimental.pallas import tpu as pltpu
from jax.experimental.pallas import tpu_sc as plsc

NB = 1024
LANES = 16


def _sigmoid(x):
    return 0.5 * jnp.tanh(0.5 * x) + 0.5


def _gru_update(gi, gh, h):
    H = h.shape[-1]
    r = _sigmoid(gi[:, :H] + gh[:, :H])
    z = _sigmoid(gi[:, H:2 * H] + gh[:, H:2 * H])
    nt = jnp.tanh(gi[:, 2 * H:] + r * gh[:, 2 * H:])
    return (1.0 - z) * nt + z * h


BLK = 1024


def _tc_gru_body(nsteps_ref, off_ref, m_ref,
                 eWihT, eWhhT, ebih, ebhh,
                 dWcat, dbih, dbhh,
                 W1T, b1, W2T, b2,
                 xs_hbm, ys_hbm,
                 slab2, oslab2, h_ref, sem_in, sem_out):
    nsteps = nsteps_ref[0]

    def in_start(k, src_off):
        pltpu.make_async_copy(
            xs_hbm.at[pl.ds(pl.multiple_of(src_off, 8), BLK)],
            slab2.at[k], sem_in.at[k]).start()

    def in_wait(k):
        pltpu.make_async_copy(
            xs_hbm.at[pl.ds(0, BLK)], slab2.at[k], sem_in.at[k]).wait()

    def out_start(k, dst_off):
        pltpu.make_async_copy(
            oslab2.at[k], ys_hbm.at[pl.ds(pl.multiple_of(dst_off, 8), BLK)],
            sem_out.at[k]).start()

    def out_wait(k):
        pltpu.make_async_copy(
            oslab2.at[k], ys_hbm.at[pl.ds(0, BLK)], sem_out.at[k]).wait()

    in_start(0, off_ref[0])
    h_ref[...] = jnp.zeros_like(h_ref)

    def enc_step(t, g):
        o_t = off_ref[t]
        nblk = (off_ref[t + 1] - o_t + BLK - 1) // BLK

        def blk(b, g):
            k = lax.rem(g, 2)
            b0 = pl.multiple_of(b * BLK, BLK)
            last = b + 1 == nblk
            nxt_off = jnp.where(last, off_ref[t + 1], o_t + b0 + BLK)

            @pl.when(jnp.logical_or(jnp.logical_not(last), t + 1 < nsteps))
            def _():
                in_start(lax.rem(g + 1, 2), nxt_off)

            in_wait(k)
            h = h_ref[pl.ds(b0, BLK)]
            gi = jnp.dot(slab2[k].astype(jnp.bfloat16), eWihT[...],
                         preferred_element_type=jnp.float32) + ebih[...]
            gh = jnp.dot(h.astype(jnp.bfloat16), eWhhT[...],
                         preferred_element_type=jnp.float32) + ebhh[...]
            hn = _gru_update(gi, gh, h)
            mask = m_ref[pl.ds(b0, BLK)] > t
            h_ref[pl.ds(b0, BLK)] = jnp.where(mask, hn, h)
            return g + 1

        return lax.fori_loop(0, nblk, blk, g)

    lax.fori_loop(0, nsteps, enc_step, 0)

    def dec_step(i, g):
        o_i = off_ref[i]
        nblk = (off_ref[i + 1] - o_i + BLK - 1) // BLK

        def blk(b, g):
            k = lax.rem(g, 2)
            b0 = pl.multiple_of(b * BLK, BLK)
            hid = h_ref[pl.ds(b0, BLK)]
            G = jnp.dot(hid.astype(jnp.bfloat16), dWcat[...],
                        preferred_element_type=jnp.float32)
            H3 = dbih.shape[-1]
            gi = jnp.where(i == 0, 0.0, G[:, :H3]) + dbih[...]
            gh = G[:, H3:] + dbhh[...]
            hidn = _gru_update(gi, gh, hid)
            h_ref[pl.ds(b0, BLK)] = hidn
            h1 = jnp.maximum(
                jnp.dot(hidn.astype(jnp.bfloat16), W1T[...],
                        preferred_element_type=jnp.float32) + b1[...], 0.0)
            xo = (jnp.dot(h1.astype(jnp.bfloat16), W2T[...],
                          preferred_element_type=jnp.float32) + b2[...])

            @pl.when(g >= 2)
            def _():
                out_wait(k)

            oslab2[k] = xo
            out_start(k, o_i + b0)
            return g + 1

        return lax.fori_loop(0, nblk, blk, g)

    gd = lax.fori_loop(0, nsteps, dec_step, 0)

    @pl.when(gd >= 1)
    def _():
        out_wait(lax.rem(gd + 1, 2))

    @pl.when(gd >= 2)
    def _():
        out_wait(lax.rem(gd, 2))


def _tc_gru(nsteps, off, m_col, weights, xs, T, dim):
    smem = pl.BlockSpec(memory_space=pltpu.SMEM)
    vmem = pl.BlockSpec(memory_space=pltpu.VMEM)
    anyspace = pl.BlockSpec(memory_space=pl.ANY)
    return pl.pallas_call(
        _tc_gru_body,
        in_specs=[smem, smem, vmem] + [vmem] * 11 + [anyspace],
        out_specs=anyspace,
        out_shape=jax.ShapeDtypeStruct((8 * T + NB, dim), jnp.float32),
        scratch_shapes=[
            pltpu.VMEM((2, BLK, dim), jnp.float32),
            pltpu.VMEM((2, BLK, dim), jnp.float32),
            pltpu.VMEM((NB, weights[1].shape[0]), jnp.float32),
            pltpu.SemaphoreType.DMA((2,)),
            pltpu.SemaphoreType.DMA((2,)),
        ],
    )(nsteps, off, m_col, *weights, xs)


@functools.cache
def _sc_kernels(T, dim):
    info = plsc.get_sparse_core_info()
    nc, ns = info.num_cores, info.num_subcores
    nw = nc * ns
    rows_w = T // nw
    CH = 128
    nch = rows_w // CH
    mesh = plsc.VectorSubcoreMesh(core_axis_name="c", subcore_axis_name="s")
    params = pltpu.CompilerParams(needs_layout_passes=False)

    AT = 8 * T + NB

    @functools.partial(
        pl.kernel, mesh=mesh, compiler_params=params,
        out_type=[jax.ShapeDtypeStruct((AT, dim), jnp.float32),
                  jax.ShapeDtypeStruct((T // CH, CH), jnp.int32)],
        scratch_types=[
            pltpu.VMEM((rows_w,), jnp.int32),
            pltpu.VMEM((NB,), jnp.int32),
            pltpu.VMEM((NB,), jnp.int32),
            pltpu.VMEM((T,), jnp.int32),
            pltpu.VMEM((nch, CH), jnp.int32),
            pltpu.VMEM((2, CH, dim), jnp.float32),
            pltpu.SemaphoreType.DMA((2,)),
            pltpu.SemaphoreType.DMA((2,)),
        ],
    )
    def sc_scatter(x_hbm, batch_hbm, ptr_hbm, r_hbm, off_hbm,
                   xs_hbm, dest_hbm,
                   b_v, ptr_v, r_v, off_v, idx2d, rows2, seml, sems):
        wid = lax.axis_index("s") * nc + lax.axis_index("c")
        base = wid * rows_w

        def load(j, k):
            return pltpu.make_async_copy(
                x_hbm.at[pl.ds(base + j * CH, CH)], rows2.at[k], seml.at[k])

        def scat(j, k):
            return pltpu.make_async_copy(
                rows2.at[k], xs_hbm.at[idx2d.at[j]], sems.at[k])

        load(0, 0).start()
        pltpu.sync_copy(batch_hbm.at[pl.ds(base, rows_w)], b_v)
        pltpu.sync_copy(ptr_hbm, ptr_v)
        pltpu.sync_copy(r_hbm, r_v)
        pltpu.sync_copy(off_hbm, off_v)
        lanes = jnp.arange(LANES, dtype=jnp.int32)
        for j in range(nch):
            for g in range(CH // LANES):
                q = j * CH + g * LANES
                s16 = b_v[pl.ds(q, LANES)]
                ptr16 = plsc.load_gather(ptr_v, [s16])
                r16 = plsc.load_gather(r_v, [s16])
                t16 = (lanes + (base + q)) - ptr16
                off16 = plsc.load_gather(off_v, [t16])
                idx2d[j, pl.ds(g * LANES, LANES)] = off16 + r16
        for j in range(nch):
            k = j % 2
            load(j, k).wait()
            scat(j, k).start()
            if j + 1 < nch:
                if j >= 1:
                    scat(j - 1, 1 - k).wait()
                load(j + 1, 1 - k).start()
        scat(nch - 2, nch % 2).wait()
        scat(nch - 1, (nch - 1) % 2).wait()
        pltpu.sync_copy(idx2d, dest_hbm.at[pl.ds(wid * nch, nch)])

    @functools.partial(
        pl.kernel, mesh=mesh, compiler_params=params,
        out_type=jax.ShapeDtypeStruct((T, dim), jnp.float32),
        scratch_types=[
            pltpu.VMEM((nch, CH), jnp.int32),
            pltpu.VMEM((2, CH, dim), jnp.float32),
            pltpu.SemaphoreType.DMA((2,)),
            pltpu.SemaphoreType.DMA((2,)),
        ],
    )
    def sc_gather(ys_hbm, dest_hbm, out_hbm, idx2d, rows2, semg, semw):
        wid = lax.axis_index("s") * nc + lax.axis_index("c")
        base = wid * rows_w

        def gath(j, k):
            return pltpu.make_async_copy(
                ys_hbm.at[idx2d.at[j]], rows2.at[k], semg.at[k])

        def store(j, k):
            return pltpu.make_async_copy(
                rows2.at[k], out_hbm.at[pl.ds(base + j * CH, CH)], semw.at[k])

        pltpu.sync_copy(dest_hbm.at[pl.ds(wid * nch, nch)], idx2d)
        gath(0, 0).start()
        for j in range(nch):
            k = j % 2
            gath(j, k).wait()
            store(j, k).start()
            if j + 1 < nch:
                if j >= 1:
                    store(j - 1, 1 - k).wait()
                gath(j + 1, 1 - k).start()
        store(nch - 2, nch % 2).wait()
        store(nch - 1, (nch - 1) % 2).wait()

    return sc_scatter, sc_gather


def kernel(x, batch, enc_Wih, enc_Whh, enc_bih, enc_bhh,
           dec_Wih, dec_Whh, dec_bih, dec_bhh,
           map_W1, map_b1, map_W2, map_b2):
    T, dim = x.shape

    sb = batch.astype(jnp.int32)
    seg = jnp.arange(NB, dtype=jnp.int32)
    ends = jnp.sum((sb[None, :] <= seg[:, None]).astype(jnp.int32), axis=1)
    ptr = jnp.concatenate([jnp.zeros((1,), jnp.int32), ends[:-1]])
    n = ends - ptr
    order = jnp.argsort(-n, stable=True).astype(jnp.int32)
    m_desc = n[order]
    rank = jnp.argsort(order).astype(jnp.int32)
    max_n = m_desc[0]
    tgrid = jnp.arange(T, dtype=jnp.int32)
    c = NB - jnp.sum((m_desc[None, :] <= tgrid[:, None]).astype(jnp.int32),
                     axis=1)
    c8 = ((c + 7) // 8) * 8
    off = jnp.concatenate([jnp.zeros((1,), jnp.int32),
                           jnp.cumsum(c8, dtype=jnp.int32)])

    sc_scatter, sc_gather = _sc_kernels(T, dim)
    xs, dest = sc_scatter(x, sb, ptr, rank, off[:T])

    bf16 = jnp.bfloat16
    weights = (
        enc_Wih.T.astype(bf16), enc_Whh.T.astype(bf16),
        enc_bih.reshape(1, -1), enc_bhh.reshape(1, -1),
        jnp.concatenate([dec_Wih.T, dec_Whh.T], axis=1).astype(bf16),
        dec_bih.reshape(1, -1), dec_bhh.reshape(1, -1),
        map_W1.T.astype(bf16), map_b1.reshape(1, -1),
        map_W2.T.astype(bf16), map_b2.reshape(1, -1),
    )
    ys = _tc_gru(max_n.reshape(1), off, m_desc[:, None], weights, xs, T, dim)
    x_flat = sc_gather(ys, dest)
    return (x_flat, batch)


# speedup vs baseline: 1.0276x; 1.0276x over previous
import functools

import jax
import jax.numpy as jnp
from jax import lax
from jax.experimental import pallas as pl
from jax.experimental.pallas import tpu as pltpu
from jax.experimental.pallas import tpu_sc as plsc

NB = 1024
LANES = 16


def _sigmoid(x):
    return 0.5 * jnp.tanh(0.5 * x) + 0.5


def _gru_update(gi, gh, h):
    H = h.shape[-1]
    r = _sigmoid(gi[:, :H] + gh[:, :H])
    z = _sigmoid(gi[:, H:2 * H] + gh[:, H:2 * H])
    nt = jnp.tanh(gi[:, 2 * H:] + r * gh[:, 2 * H:])
    return (1.0 - z) * nt + z * h


BLK = 1024


def _tc_gru_body(nsteps_ref, off_ref, m_ref,
                 eWihT, eWhhT, ebih, ebhh,
                 dWcat, dbih, dbhh,
                 W1T, b1, W2T, b2,
                 xs_hbm, ys_hbm,
                 slab2, oslab2, h_ref, sem_in, sem_out):
    nsteps = nsteps_ref[0]

    def in_start(k, src_off):
        pltpu.make_async_copy(
            xs_hbm.at[pl.ds(pl.multiple_of(src_off, 8), BLK)],
            slab2.at[k], sem_in.at[k]).start()

    def in_wait(k):
        pltpu.make_async_copy(
            xs_hbm.at[pl.ds(0, BLK)], slab2.at[k], sem_in.at[k]).wait()

    def out_start(k, dst_off):
        pltpu.make_async_copy(
            oslab2.at[k], ys_hbm.at[pl.ds(pl.multiple_of(dst_off, 8), BLK)],
            sem_out.at[k]).start()

    def out_wait(k):
        pltpu.make_async_copy(
            oslab2.at[k], ys_hbm.at[pl.ds(0, BLK)], sem_out.at[k]).wait()

    in_start(0, off_ref[0])
    h_ref[...] = jnp.zeros_like(h_ref)

    def enc_step(t, g):
        o_t = off_ref[t]
        nblk = (off_ref[t + 1] - o_t + BLK - 1) // BLK

        def blk(b, g):
            k = lax.rem(g, 2)
            b0 = pl.multiple_of(b * BLK, BLK)
            last = b + 1 == nblk
            nxt_off = jnp.where(last, off_ref[t + 1], o_t + b0 + BLK)

            @pl.when(jnp.logical_or(jnp.logical_not(last), t + 1 < nsteps))
            def _():
                in_start(lax.rem(g + 1, 2), nxt_off)

            in_wait(k)
            h = h_ref[pl.ds(b0, BLK)]
            gi = jnp.dot(slab2[k].astype(jnp.bfloat16), eWihT[...],
                         preferred_element_type=jnp.float32) + ebih[...]
            gh = jnp.dot(h.astype(jnp.bfloat16), eWhhT[...],
                         preferred_element_type=jnp.float32) + ebhh[...]
            hn = _gru_update(gi, gh, h)
            mask = m_ref[pl.ds(b0, BLK)] > t
            h_ref[pl.ds(b0, BLK)] = jnp.where(mask, hn, h)
            return g + 1

        return lax.fori_loop(0, nblk, blk, g)

    lax.fori_loop(0, nsteps, enc_step, 0)

    def dec_step(i, g):
        o_i = off_ref[i]
        nblk = (off_ref[i + 1] - o_i + BLK - 1) // BLK

        def blk(b, g):
            k = lax.rem(g, 2)
            b0 = pl.multiple_of(b * BLK, BLK)
            hid = h_ref[pl.ds(b0, BLK)]
            hb = hid.astype(jnp.bfloat16)
            curr = jnp.where(i == 0, jnp.zeros_like(hb), hb)
            H3 = dbih.shape[-1]
            gi = jnp.dot(curr, dWcat[:, :H3],
                         preferred_element_type=jnp.float32) + dbih[...]
            gh = jnp.dot(hb, dWcat[:, H3:],
                         preferred_element_type=jnp.float32) + dbhh[...]
            hidn = _gru_update(gi, gh, hid)
            h_ref[pl.ds(b0, BLK)] = hidn
            h1 = jnp.maximum(
                jnp.dot(hidn.astype(jnp.bfloat16), W1T[...],
                        preferred_element_type=jnp.float32) + b1[...], 0.0)
            xo = (jnp.dot(h1.astype(jnp.bfloat16), W2T[...],
                          preferred_element_type=jnp.float32) + b2[...])

            @pl.when(g >= 2)
            def _():
                out_wait(k)

            oslab2[k] = xo
            out_start(k, o_i + b0)
            return g + 1

        return lax.fori_loop(0, nblk, blk, g)

    gd = lax.fori_loop(0, nsteps, dec_step, 0)

    @pl.when(gd >= 1)
    def _():
        out_wait(lax.rem(gd + 1, 2))

    @pl.when(gd >= 2)
    def _():
        out_wait(lax.rem(gd, 2))


def _tc_gru(nsteps, off, m_col, weights, xs, T, dim):
    smem = pl.BlockSpec(memory_space=pltpu.SMEM)
    vmem = pl.BlockSpec(memory_space=pltpu.VMEM)
    anyspace = pl.BlockSpec(memory_space=pl.ANY)
    return pl.pallas_call(
        _tc_gru_body,
        in_specs=[smem, smem, vmem] + [vmem] * 11 + [anyspace],
        out_specs=anyspace,
        out_shape=jax.ShapeDtypeStruct((8 * T + NB, dim), jnp.float32),
        scratch_shapes=[
            pltpu.VMEM((2, BLK, dim), jnp.float32),
            pltpu.VMEM((2, BLK, dim), jnp.float32),
            pltpu.VMEM((NB, weights[1].shape[0]), jnp.float32),
            pltpu.SemaphoreType.DMA((2,)),
            pltpu.SemaphoreType.DMA((2,)),
        ],
    )(nsteps, off, m_col, *weights, xs)


@functools.cache
def _sc_kernels(T, dim):
    info = plsc.get_sparse_core_info()
    nc, ns = info.num_cores, info.num_subcores
    nw = nc * ns
    rows_w = T // nw
    CH = 128
    nch = rows_w // CH
    mesh = plsc.VectorSubcoreMesh(core_axis_name="c", subcore_axis_name="s")
    params = pltpu.CompilerParams(needs_layout_passes=False)

    AT = 8 * T + NB

    @functools.partial(
        pl.kernel, mesh=mesh, compiler_params=params,
        out_type=[jax.ShapeDtypeStruct((AT, dim), jnp.float32),
                  jax.ShapeDtypeStruct((T // CH, CH), jnp.int32)],
        scratch_types=[
            pltpu.VMEM((rows_w,), jnp.int32),
            pltpu.VMEM((NB,), jnp.int32),
            pltpu.VMEM((NB,), jnp.int32),
            pltpu.VMEM((T,), jnp.int32),
            pltpu.VMEM((nch, CH), jnp.int32),
            pltpu.VMEM((2, CH, dim), jnp.float32),
            pltpu.SemaphoreType.DMA((2,)),
            pltpu.SemaphoreType.DMA((2,)),
        ],
    )
    def sc_scatter(x_hbm, batch_hbm, ptr_hbm, r_hbm, off_hbm,
                   xs_hbm, dest_hbm,
                   b_v, ptr_v, r_v, off_v, idx2d, rows2, seml, sems):
        wid = lax.axis_index("s") * nc + lax.axis_index("c")
        base = wid * rows_w

        def load(j, k):
            return pltpu.make_async_copy(
                x_hbm.at[pl.ds(base + j * CH, CH)], rows2.at[k], seml.at[k])

        def scat(j, k):
            return pltpu.make_async_copy(
                rows2.at[k], xs_hbm.at[idx2d.at[j]], sems.at[k])

        load(0, 0).start()
        pltpu.sync_copy(batch_hbm.at[pl.ds(base, rows_w)], b_v)
        pltpu.sync_copy(ptr_hbm, ptr_v)
        pltpu.sync_copy(r_hbm, r_v)
        pltpu.sync_copy(off_hbm, off_v)
        lanes = jnp.arange(LANES, dtype=jnp.int32)
        for j in range(nch):
            for g in range(CH // LANES):
                q = j * CH + g * LANES
                s16 = b_v[pl.ds(q, LANES)]
                ptr16 = plsc.load_gather(ptr_v, [s16])
                r16 = plsc.load_gather(r_v, [s16])
                t16 = (lanes + (base + q)) - ptr16
                off16 = plsc.load_gather(off_v, [t16])
                idx2d[j, pl.ds(g * LANES, LANES)] = off16 + r16
        for j in range(nch):
            k = j % 2
            load(j, k).wait()
            scat(j, k).start()
            if j + 1 < nch:
                if j >= 1:
                    scat(j - 1, 1 - k).wait()
                load(j + 1, 1 - k).start()
        scat(nch - 2, nch % 2).wait()
        scat(nch - 1, (nch - 1) % 2).wait()
        pltpu.sync_copy(idx2d, dest_hbm.at[pl.ds(wid * nch, nch)])

    @functools.partial(
        pl.kernel, mesh=mesh, compiler_params=params,
        out_type=jax.ShapeDtypeStruct((T, dim), jnp.float32),
        scratch_types=[
            pltpu.VMEM((nch, CH), jnp.int32),
            pltpu.VMEM((2, CH, dim), jnp.float32),
            pltpu.SemaphoreType.DMA((2,)),
            pltpu.SemaphoreType.DMA((2,)),
        ],
    )
    def sc_gather(ys_hbm, dest_hbm, out_hbm, idx2d, rows2, semg, semw):
        wid = lax.axis_index("s") * nc + lax.axis_index("c")
        base = wid * rows_w

        def gath(j, k):
            return pltpu.make_async_copy(
                ys_hbm.at[idx2d.at[j]], rows2.at[k], semg.at[k])

        def store(j, k):
            return pltpu.make_async_copy(
                rows2.at[k], out_hbm.at[pl.ds(base + j * CH, CH)], semw.at[k])

        pltpu.sync_copy(dest_hbm.at[pl.ds(wid * nch, nch)], idx2d)
        gath(0, 0).start()
        for j in range(nch):
            k = j % 2
            gath(j, k).wait()
            store(j, k).start()
            if j + 1 < nch:
                if j >= 1:
                    store(j - 1, 1 - k).wait()
                gath(j + 1, 1 - k).start()
        store(nch - 2, nch % 2).wait()
        store(nch - 1, (nch - 1) % 2).wait()

    return sc_scatter, sc_gather


def kernel(x, batch, enc_Wih, enc_Whh, enc_bih, enc_bhh,
           dec_Wih, dec_Whh, dec_bih, dec_bhh,
           map_W1, map_b1, map_W2, map_b2):
    T, dim = x.shape

    sb = batch.astype(jnp.int32)
    seg = jnp.arange(NB, dtype=jnp.int32)
    ends = jnp.sum((sb[None, :] <= seg[:, None]).astype(jnp.int32), axis=1)
    ptr = jnp.concatenate([jnp.zeros((1,), jnp.int32), ends[:-1]])
    n = ends - ptr
    order = jnp.argsort(-n, stable=True).astype(jnp.int32)
    m_desc = n[order]
    rank = jnp.argsort(order).astype(jnp.int32)
    max_n = m_desc[0]
    tgrid = jnp.arange(T, dtype=jnp.int32)
    c = NB - jnp.sum((m_desc[None, :] <= tgrid[:, None]).astype(jnp.int32),
                     axis=1)
    c8 = ((c + 7) // 8) * 8
    off = jnp.concatenate([jnp.zeros((1,), jnp.int32),
                           jnp.cumsum(c8, dtype=jnp.int32)])

    sc_scatter, sc_gather = _sc_kernels(T, dim)
    xs, dest = sc_scatter(x, sb, ptr, rank, off[:T])

    bf16 = jnp.bfloat16
    weights = (
        enc_Wih.T.astype(bf16), enc_Whh.T.astype(bf16),
        enc_bih.reshape(1, -1), enc_bhh.reshape(1, -1),
        jnp.concatenate([dec_Wih.T, dec_Whh.T], axis=1).astype(bf16),
        dec_bih.reshape(1, -1), dec_bhh.reshape(1, -1),
        map_W1.T.astype(bf16), map_b1.reshape(1, -1),
        map_W2.T.astype(bf16), map_b2.reshape(1, -1),
    )
    ys = _tc_gru(max_n.reshape(1), off, m_desc[:, None], weights, xs, T, dim)
    x_flat = sc_gather(ys, dest)
    return (x_flat, batch)


# speedup vs baseline: 1.1663x; 1.1349x over previous
import functools

import jax
import jax.numpy as jnp
from jax import lax
from jax.experimental import pallas as pl
from jax.experimental.pallas import tpu as pltpu
from jax.experimental.pallas import tpu_sc as plsc

NB = 1024
LANES = 16


def _sigmoid(x):
    return 0.5 * jnp.tanh(0.5 * x) + 0.5


def _gru_update(gi, gh, h):
    H = h.shape[-1]
    r = _sigmoid(gi[:, :H] + gh[:, :H])
    z = _sigmoid(gi[:, H:2 * H] + gh[:, H:2 * H])
    nt = jnp.tanh(gi[:, 2 * H:] + r * gh[:, 2 * H:])
    return (1.0 - z) * nt + z * h


BLK = 1024


def _tc_gru_body(nsteps_ref, off_ref, m_ref,
                 eWihT, eWhhT, ebih, ebhh,
                 dWcat, dbih, dbhh,
                 W1T, b1, W2T, b2,
                 xs_hbm, ys_hbm,
                 slab2, oslab2, h_ref, sem_in, sem_out):
    nsteps = nsteps_ref[0]

    def in_start(k, src_off):
        pltpu.make_async_copy(
            xs_hbm.at[pl.ds(pl.multiple_of(src_off, 8), BLK)],
            slab2.at[k], sem_in.at[k]).start()

    def in_wait(k):
        pltpu.make_async_copy(
            xs_hbm.at[pl.ds(0, BLK)], slab2.at[k], sem_in.at[k]).wait()

    def out_start(k, dst_off):
        pltpu.make_async_copy(
            oslab2.at[k], ys_hbm.at[pl.ds(pl.multiple_of(dst_off, 8), BLK)],
            sem_out.at[k]).start()

    def out_wait(k):
        pltpu.make_async_copy(
            oslab2.at[k], ys_hbm.at[pl.ds(0, BLK)], sem_out.at[k]).wait()

    in_start(0, off_ref[0])
    h_ref[...] = jnp.zeros_like(h_ref)

    def enc_step(t, g):
        o_t = off_ref[t]
        nblk = (off_ref[t + 1] - o_t + BLK - 1) // BLK

        def blk(b, g):
            k = lax.rem(g, 2)
            b0 = pl.multiple_of(b * BLK, BLK)
            last = b + 1 == nblk
            nxt_off = jnp.where(last, off_ref[t + 1], o_t + b0 + BLK)

            @pl.when(jnp.logical_or(jnp.logical_not(last), t + 1 < nsteps))
            def _():
                in_start(lax.rem(g + 1, 2), nxt_off)

            in_wait(k)
            h = h_ref[pl.ds(b0, BLK)]
            gi = jnp.dot(slab2[k].astype(jnp.bfloat16), eWihT[...],
                         preferred_element_type=jnp.float32) + ebih[...]
            gh = jnp.dot(h.astype(jnp.bfloat16), eWhhT[...],
                         preferred_element_type=jnp.float32) + ebhh[...]
            hn = _gru_update(gi, gh, h)
            mask = m_ref[pl.ds(b0, BLK)] > t
            h_ref[pl.ds(b0, BLK)] = jnp.where(mask, hn, h)
            return g + 1

        return lax.fori_loop(0, nblk, blk, g)

    lax.fori_loop(0, nsteps, enc_step, 0)

    def dec_step(i, g):
        o_i = off_ref[i]
        nblk = (off_ref[i + 1] - o_i + BLK - 1) // BLK

        def blk(b, g):
            k = lax.rem(g, 2)
            b0 = pl.multiple_of(b * BLK, BLK)
            hid = h_ref[pl.ds(b0, BLK)]
            hb = hid.astype(jnp.bfloat16)
            curr = jnp.where(i == 0, jnp.zeros_like(hb), hb)
            H3 = dbih.shape[-1]
            gi = jnp.dot(curr, dWcat[:, :H3],
                         preferred_element_type=jnp.float32) + dbih[...]
            gh = jnp.dot(hb, dWcat[:, H3:],
                         preferred_element_type=jnp.float32) + dbhh[...]
            hidn = _gru_update(gi, gh, hid)
            h_ref[pl.ds(b0, BLK)] = hidn
            h1 = jnp.maximum(
                jnp.dot(hidn.astype(jnp.bfloat16), W1T[...],
                        preferred_element_type=jnp.float32) + b1[...], 0.0)
            xo = (jnp.dot(h1.astype(jnp.bfloat16), W2T[...],
                          preferred_element_type=jnp.float32) + b2[...])

            @pl.when(g >= 2)
            def _():
                out_wait(k)

            oslab2[k] = xo
            out_start(k, o_i + b0)
            return g + 1

        return lax.fori_loop(0, nblk, blk, g)

    gd = lax.fori_loop(0, nsteps, dec_step, 0)

    @pl.when(gd >= 1)
    def _():
        out_wait(lax.rem(gd + 1, 2))

    @pl.when(gd >= 2)
    def _():
        out_wait(lax.rem(gd, 2))


def _tc_gru(nsteps, off, m_col, weights, xs, T, dim):
    smem = pl.BlockSpec(memory_space=pltpu.SMEM)
    vmem = pl.BlockSpec(memory_space=pltpu.VMEM)
    anyspace = pl.BlockSpec(memory_space=pl.ANY)
    return pl.pallas_call(
        _tc_gru_body,
        in_specs=[smem, smem, vmem] + [vmem] * 11 + [anyspace],
        out_specs=anyspace,
        out_shape=jax.ShapeDtypeStruct((8 * T + NB, dim), jnp.float32),
        scratch_shapes=[
            pltpu.VMEM((2, BLK, dim), jnp.float32),
            pltpu.VMEM((2, BLK, dim), jnp.float32),
            pltpu.VMEM((NB, weights[1].shape[0]), jnp.float32),
            pltpu.SemaphoreType.DMA((2,)),
            pltpu.SemaphoreType.DMA((2,)),
        ],
    )(nsteps, off, m_col, *weights, xs)


@functools.cache
def _sc_kernels(T, dim):
    info = plsc.get_sparse_core_info()
    nc, ns = info.num_cores, info.num_subcores
    nw = nc * ns
    rows_w = T // nw
    CH = 128
    nch = rows_w // CH
    mesh = plsc.VectorSubcoreMesh(core_axis_name="c", subcore_axis_name="s")
    params = pltpu.CompilerParams(needs_layout_passes=False)

    AT = 8 * T + NB

    @functools.partial(
        pl.kernel, mesh=mesh, compiler_params=params,
        out_type=[jax.ShapeDtypeStruct((AT, dim), jnp.float32),
                  jax.ShapeDtypeStruct((T // CH, CH), jnp.int32)],
        scratch_types=[
            pltpu.VMEM((rows_w,), jnp.int32),
            pltpu.VMEM((NB,), jnp.int32),
            pltpu.VMEM((NB,), jnp.int32),
            pltpu.VMEM((T,), jnp.int32),
            pltpu.VMEM((nch, CH), jnp.int32),
            pltpu.VMEM((2, CH, dim), jnp.float32),
            pltpu.SemaphoreType.DMA((2,)),
            pltpu.SemaphoreType.DMA((2,)),
        ],
    )
    def sc_scatter(x_hbm, batch_hbm, ptr_hbm, r_hbm, off_hbm,
                   xs_hbm, dest_hbm,
                   b_v, ptr_v, r_v, off_v, idx2d, rows2, seml, sems):
        wid = lax.axis_index("s") * nc + lax.axis_index("c")
        base = wid * rows_w

        def load(j, k):
            return pltpu.make_async_copy(
                x_hbm.at[pl.ds(base + j * CH, CH)], rows2.at[k], seml.at[k])

        def scat(j, k):
            return pltpu.make_async_copy(
                rows2.at[k], xs_hbm.at[idx2d.at[j]], sems.at[k])

        load(0, 0).start()
        pltpu.sync_copy(batch_hbm.at[pl.ds(base, rows_w)], b_v)
        pltpu.sync_copy(ptr_hbm, ptr_v)
        pltpu.sync_copy(r_hbm, r_v)
        pltpu.sync_copy(off_hbm, off_v)
        lanes = jnp.arange(LANES, dtype=jnp.int32)
        for j in range(nch):
            for g in range(CH // LANES):
                q = j * CH + g * LANES
                s16 = b_v[pl.ds(q, LANES)]
                ptr16 = plsc.load_gather(ptr_v, [s16])
                r16 = plsc.load_gather(r_v, [s16])
                t16 = (lanes + (base + q)) - ptr16
                off16 = plsc.load_gather(off_v, [t16])
                idx2d[j, pl.ds(g * LANES, LANES)] = off16 + r16
        for j in range(nch):
            k = j % 2
            load(j, k).wait()
            scat(j, k).start()
            if j + 1 < nch:
                if j >= 1:
                    scat(j - 1, 1 - k).wait()
                load(j + 1, 1 - k).start()
        scat(nch - 2, nch % 2).wait()
        scat(nch - 1, (nch - 1) % 2).wait()
        pltpu.sync_copy(idx2d, dest_hbm.at[pl.ds(wid * nch, nch)])

    @functools.partial(
        pl.kernel, mesh=mesh, compiler_params=params,
        out_type=jax.ShapeDtypeStruct((T, dim), jnp.float32),
        scratch_types=[
            pltpu.VMEM((nch, CH), jnp.int32),
            pltpu.VMEM((2, CH, dim), jnp.float32),
            pltpu.SemaphoreType.DMA((2,)),
            pltpu.SemaphoreType.DMA((2,)),
        ],
    )
    def sc_gather(ys_hbm, dest_hbm, out_hbm, idx2d, rows2, semg, semw):
        wid = lax.axis_index("s") * nc + lax.axis_index("c")
        base = wid * rows_w

        def gath(j, k):
            return pltpu.make_async_copy(
                ys_hbm.at[idx2d.at[j]], rows2.at[k], semg.at[k])

        def store(j, k):
            return pltpu.make_async_copy(
                rows2.at[k], out_hbm.at[pl.ds(base + j * CH, CH)], semw.at[k])

        pltpu.sync_copy(dest_hbm.at[pl.ds(wid * nch, nch)], idx2d)
        gath(0, 0).start()
        for j in range(nch):
            k = j % 2
            gath(j, k).wait()
            store(j, k).start()
            if j + 1 < nch:
                if j >= 1:
                    store(j - 1, 1 - k).wait()
                gath(j + 1, 1 - k).start()
        store(nch - 2, nch % 2).wait()
        store(nch - 1, (nch - 1) % 2).wait()

    return sc_scatter, sc_gather


def kernel(x, batch, enc_Wih, enc_Whh, enc_bih, enc_bhh,
           dec_Wih, dec_Whh, dec_bih, dec_bhh,
           map_W1, map_b1, map_W2, map_b2):
    T, dim = x.shape

    sb = batch.astype(jnp.int32)
    seg = jnp.arange(NB, dtype=jnp.int32)
    ends = jnp.sum((sb[:, None] <= seg[None, :]).astype(jnp.int32), axis=0)
    ptr = jnp.concatenate([jnp.zeros((1,), jnp.int32), ends[:-1]])
    n = ends - ptr
    order = jnp.argsort(-n, stable=True).astype(jnp.int32)
    m_desc = n[order]
    rank = jnp.argsort(order).astype(jnp.int32)
    max_n = m_desc[0]
    TCUT, TOPK = 2048, 16
    t_lo = jnp.arange(TCUT, dtype=jnp.int32)
    c_lo = NB - jnp.sum((m_desc[None, :] <= t_lo[:, None]).astype(jnp.int32),
                        axis=1)
    t_hi = jnp.arange(TCUT, T, dtype=jnp.int32)
    c_hi = jnp.sum((m_desc[None, :TOPK] > t_hi[:, None]).astype(jnp.int32),
                   axis=1)
    c = jnp.concatenate([c_lo, c_hi])
    c8 = ((c + 7) // 8) * 8
    off = jnp.concatenate([jnp.zeros((1,), jnp.int32),
                           jnp.cumsum(c8, dtype=jnp.int32)])

    sc_scatter, sc_gather = _sc_kernels(T, dim)
    xs, dest = sc_scatter(x, sb, ptr, rank, off[:T])

    bf16 = jnp.bfloat16
    weights = (
        enc_Wih.T.astype(bf16), enc_Whh.T.astype(bf16),
        enc_bih.reshape(1, -1), enc_bhh.reshape(1, -1),
        jnp.concatenate([dec_Wih.T, dec_Whh.T], axis=1).astype(bf16),
        dec_bih.reshape(1, -1), dec_bhh.reshape(1, -1),
        map_W1.T.astype(bf16), map_b1.reshape(1, -1),
        map_W2.T.astype(bf16), map_b2.reshape(1, -1),
    )
    ys = _tc_gru(max_n.reshape(1), off, m_desc[:, None], weights, xs, T, dim)
    x_flat = sc_gather(ys, dest)
    return (x_flat, batch)


# speedup vs baseline: 1.1672x; 1.0008x over previous
import functools

import jax
import jax.numpy as jnp
from jax import lax
from jax.experimental import pallas as pl
from jax.experimental.pallas import tpu as pltpu
from jax.experimental.pallas import tpu_sc as plsc

NB = 1024
LANES = 16


def _gru_update(gi, gh, h):
    H = h.shape[-1]
    r = jax.nn.sigmoid(gi[:, :H] + gh[:, :H])
    z = jax.nn.sigmoid(gi[:, H:2 * H] + gh[:, H:2 * H])
    nt = jnp.tanh(gi[:, 2 * H:] + r * gh[:, 2 * H:])
    return (1.0 - z) * nt + z * h


BLK = 1024


def _tc_gru_body(nsteps_ref, off_ref, m_ref,
                 eWihT, eWhhT, ebih, ebhh,
                 dWcat, dbih, dbhh,
                 W1T, b1, W2T, b2,
                 xs_hbm, ys_hbm,
                 slab2, oslab2, h_ref, sem_in, sem_out):
    nsteps = nsteps_ref[0]

    def in_start(k, src_off):
        pltpu.make_async_copy(
            xs_hbm.at[pl.ds(pl.multiple_of(src_off, 8), BLK)],
            slab2.at[k], sem_in.at[k]).start()

    def in_wait(k):
        pltpu.make_async_copy(
            xs_hbm.at[pl.ds(0, BLK)], slab2.at[k], sem_in.at[k]).wait()

    def out_start(k, dst_off):
        pltpu.make_async_copy(
            oslab2.at[k], ys_hbm.at[pl.ds(pl.multiple_of(dst_off, 8), BLK)],
            sem_out.at[k]).start()

    def out_wait(k):
        pltpu.make_async_copy(
            oslab2.at[k], ys_hbm.at[pl.ds(0, BLK)], sem_out.at[k]).wait()

    in_start(0, off_ref[0])
    h_ref[...] = jnp.zeros_like(h_ref)

    def enc_step(t, g):
        o_t = off_ref[t]
        nblk = (off_ref[t + 1] - o_t + BLK - 1) // BLK

        def blk(b, g):
            k = lax.rem(g, 2)
            b0 = pl.multiple_of(b * BLK, BLK)
            last = b + 1 == nblk
            nxt_off = jnp.where(last, off_ref[t + 1], o_t + b0 + BLK)

            @pl.when(jnp.logical_or(jnp.logical_not(last), t + 1 < nsteps))
            def _():
                in_start(lax.rem(g + 1, 2), nxt_off)

            in_wait(k)
            h = h_ref[pl.ds(b0, BLK)]
            gi = jnp.dot(slab2[k].astype(jnp.bfloat16), eWihT[...],
                         preferred_element_type=jnp.float32) + ebih[...]
            gh = jnp.dot(h.astype(jnp.bfloat16), eWhhT[...],
                         preferred_element_type=jnp.float32) + ebhh[...]
            hn = _gru_update(gi, gh, h)
            mask = m_ref[pl.ds(b0, BLK)] > t
            h_ref[pl.ds(b0, BLK)] = jnp.where(mask, hn, h)
            return g + 1

        return lax.fori_loop(0, nblk, blk, g)

    lax.fori_loop(0, nsteps, enc_step, 0)

    def dec_step(i, g):
        o_i = off_ref[i]
        nblk = (off_ref[i + 1] - o_i + BLK - 1) // BLK

        def blk(b, g):
            k = lax.rem(g, 2)
            b0 = pl.multiple_of(b * BLK, BLK)
            hid = h_ref[pl.ds(b0, BLK)]
            hb = hid.astype(jnp.bfloat16)
            curr = jnp.where(i == 0, jnp.zeros_like(hb), hb)
            H3 = dbih.shape[-1]
            gi = jnp.dot(curr, dWcat[:, :H3],
                         preferred_element_type=jnp.float32) + dbih[...]
            gh = jnp.dot(hb, dWcat[:, H3:],
                         preferred_element_type=jnp.float32) + dbhh[...]
            hidn = _gru_update(gi, gh, hid)
            h_ref[pl.ds(b0, BLK)] = hidn
            h1 = jnp.maximum(
                jnp.dot(hidn.astype(jnp.bfloat16), W1T[...],
                        preferred_element_type=jnp.float32) + b1[...], 0.0)
            xo = (jnp.dot(h1.astype(jnp.bfloat16), W2T[...],
                          preferred_element_type=jnp.float32) + b2[...])

            @pl.when(g >= 2)
            def _():
                out_wait(k)

            oslab2[k] = xo
            out_start(k, o_i + b0)
            return g + 1

        return lax.fori_loop(0, nblk, blk, g)

    gd = lax.fori_loop(0, nsteps, dec_step, 0)

    @pl.when(gd >= 1)
    def _():
        out_wait(lax.rem(gd + 1, 2))

    @pl.when(gd >= 2)
    def _():
        out_wait(lax.rem(gd, 2))


def _tc_gru(nsteps, off, m_col, weights, xs, T, dim):
    smem = pl.BlockSpec(memory_space=pltpu.SMEM)
    vmem = pl.BlockSpec(memory_space=pltpu.VMEM)
    anyspace = pl.BlockSpec(memory_space=pl.ANY)
    return pl.pallas_call(
        _tc_gru_body,
        in_specs=[smem, smem, vmem] + [vmem] * 11 + [anyspace],
        out_specs=anyspace,
        out_shape=jax.ShapeDtypeStruct((8 * T + NB, dim), jnp.float32),
        scratch_shapes=[
            pltpu.VMEM((2, BLK, dim), jnp.float32),
            pltpu.VMEM((2, BLK, dim), jnp.float32),
            pltpu.VMEM((NB, weights[1].shape[0]), jnp.float32),
            pltpu.SemaphoreType.DMA((2,)),
            pltpu.SemaphoreType.DMA((2,)),
        ],
    )(nsteps, off, m_col, *weights, xs)


@functools.cache
def _sc_kernels(T, dim):
    info = plsc.get_sparse_core_info()
    nc, ns = info.num_cores, info.num_subcores
    nw = nc * ns
    rows_w = T // nw
    CH = 128
    nch = rows_w // CH
    mesh = plsc.VectorSubcoreMesh(core_axis_name="c", subcore_axis_name="s")
    params = pltpu.CompilerParams(needs_layout_passes=False)

    AT = 8 * T + NB

    @functools.partial(
        pl.kernel, mesh=mesh, compiler_params=params,
        out_type=[jax.ShapeDtypeStruct((AT, dim), jnp.float32),
                  jax.ShapeDtypeStruct((T // CH, CH), jnp.int32)],
        scratch_types=[
            pltpu.VMEM((rows_w,), jnp.int32),
            pltpu.VMEM((NB,), jnp.int32),
            pltpu.VMEM((NB,), jnp.int32),
            pltpu.VMEM((T,), jnp.int32),
            pltpu.VMEM((nch, CH), jnp.int32),
            pltpu.VMEM((2, CH, dim), jnp.float32),
            pltpu.SemaphoreType.DMA((2,)),
            pltpu.SemaphoreType.DMA((2,)),
        ],
    )
    def sc_scatter(x_hbm, batch_hbm, ptr_hbm, r_hbm, off_hbm,
                   xs_hbm, dest_hbm,
                   b_v, ptr_v, r_v, off_v, idx2d, rows2, seml, sems):
        wid = lax.axis_index("s") * nc + lax.axis_index("c")
        base = wid * rows_w

        def load(j, k):
            return pltpu.make_async_copy(
                x_hbm.at[pl.ds(base + j * CH, CH)], rows2.at[k], seml.at[k])

        def scat(j, k):
            return pltpu.make_async_copy(
                rows2.at[k], xs_hbm.at[idx2d.at[j]], sems.at[k])

        load(0, 0).start()
        pltpu.sync_copy(batch_hbm.at[pl.ds(base, rows_w)], b_v)
        pltpu.sync_copy(ptr_hbm, ptr_v)
        pltpu.sync_copy(r_hbm, r_v)
        pltpu.sync_copy(off_hbm, off_v)
        lanes = jnp.arange(LANES, dtype=jnp.int32)
        for j in range(nch):
            for g in range(CH // LANES):
                q = j * CH + g * LANES
                s16 = b_v[pl.ds(q, LANES)]
                ptr16 = plsc.load_gather(ptr_v, [s16])
                r16 = plsc.load_gather(r_v, [s16])
                t16 = (lanes + (base + q)) - ptr16
                off16 = plsc.load_gather(off_v, [t16])
                idx2d[j, pl.ds(g * LANES, LANES)] = off16 + r16
        for j in range(nch):
            k = j % 2
            load(j, k).wait()
            scat(j, k).start()
            if j + 1 < nch:
                if j >= 1:
                    scat(j - 1, 1 - k).wait()
                load(j + 1, 1 - k).start()
        scat(nch - 2, nch % 2).wait()
        scat(nch - 1, (nch - 1) % 2).wait()
        pltpu.sync_copy(idx2d, dest_hbm.at[pl.ds(wid * nch, nch)])

    @functools.partial(
        pl.kernel, mesh=mesh, compiler_params=params,
        out_type=jax.ShapeDtypeStruct((T, dim), jnp.float32),
        scratch_types=[
            pltpu.VMEM((nch, CH), jnp.int32),
            pltpu.VMEM((2, CH, dim), jnp.float32),
            pltpu.SemaphoreType.DMA((2,)),
            pltpu.SemaphoreType.DMA((2,)),
        ],
    )
    def sc_gather(ys_hbm, dest_hbm, out_hbm, idx2d, rows2, semg, semw):
        wid = lax.axis_index("s") * nc + lax.axis_index("c")
        base = wid * rows_w

        def gath(j, k):
            return pltpu.make_async_copy(
                ys_hbm.at[idx2d.at[j]], rows2.at[k], semg.at[k])

        def store(j, k):
            return pltpu.make_async_copy(
                rows2.at[k], out_hbm.at[pl.ds(base + j * CH, CH)], semw.at[k])

        pltpu.sync_copy(dest_hbm.at[pl.ds(wid * nch, nch)], idx2d)
        gath(0, 0).start()
        for j in range(nch):
            k = j % 2
            gath(j, k).wait()
            store(j, k).start()
            if j + 1 < nch:
                if j >= 1:
                    store(j - 1, 1 - k).wait()
                gath(j + 1, 1 - k).start()
        store(nch - 2, nch % 2).wait()
        store(nch - 1, (nch - 1) % 2).wait()

    return sc_scatter, sc_gather


def kernel(x, batch, enc_Wih, enc_Whh, enc_bih, enc_bhh,
           dec_Wih, dec_Whh, dec_bih, dec_bhh,
           map_W1, map_b1, map_W2, map_b2):
    T, dim = x.shape

    sb = batch.astype(jnp.int32)
    seg = jnp.arange(NB, dtype=jnp.int32)
    ends = jnp.sum((sb[:, None] <= seg[None, :]).astype(jnp.int32), axis=0)
    ptr = jnp.concatenate([jnp.zeros((1,), jnp.int32), ends[:-1]])
    n = ends - ptr
    order = jnp.argsort(-n, stable=True).astype(jnp.int32)
    m_desc = n[order]
    rank = jnp.argsort(order).astype(jnp.int32)
    max_n = m_desc[0]
    TCUT, TOPK = 2048, 16
    t_lo = jnp.arange(TCUT, dtype=jnp.int32)
    c_lo = NB - jnp.sum((m_desc[None, :] <= t_lo[:, None]).astype(jnp.int32),
                        axis=1)
    t_hi = jnp.arange(TCUT, T, dtype=jnp.int32)
    c_hi = jnp.sum((m_desc[None, :TOPK] > t_hi[:, None]).astype(jnp.int32),
                   axis=1)
    c = jnp.concatenate([c_lo, c_hi])
    c8 = ((c + 7) // 8) * 8
    off = jnp.concatenate([jnp.zeros((1,), jnp.int32),
                           jnp.cumsum(c8, dtype=jnp.int32)])

    sc_scatter, sc_gather = _sc_kernels(T, dim)
    xs, dest = sc_scatter(x, sb, ptr, rank, off[:T])

    bf16 = jnp.bfloat16
    weights = (
        enc_Wih.T.astype(bf16), enc_Whh.T.astype(bf16),
        enc_bih.reshape(1, -1), enc_bhh.reshape(1, -1),
        jnp.concatenate([dec_Wih.T, dec_Whh.T], axis=1).astype(bf16),
        dec_bih.reshape(1, -1), dec_bhh.reshape(1, -1),
        map_W1.T.astype(bf16), map_b1.reshape(1, -1),
        map_W2.T.astype(bf16), map_b2.reshape(1, -1),
    )
    ys = _tc_gru(max_n.reshape(1), off, m_desc[:, None], weights, xs, T, dim)
    x_flat = sc_gather(ys, dest)
    return (x_flat, batch)


# speedup vs baseline: 1.1675x; 1.0003x over previous
import functools

import jax
import jax.numpy as jnp
from jax import lax
from jax.experimental import pallas as pl
from jax.experimental.pallas import tpu as pltpu
from jax.experimental.pallas import tpu_sc as plsc

NB = 1024
LANES = 16


def _gru_update(gi, gh, h):
    H = h.shape[-1]
    r = jax.nn.sigmoid(gi[:, :H] + gh[:, :H])
    z = jax.nn.sigmoid(gi[:, H:2 * H] + gh[:, H:2 * H])
    nt = jnp.tanh(gi[:, 2 * H:] + r * gh[:, 2 * H:])
    return (1.0 - z) * nt + z * h


BLK = 1024


def _tc_gru_body(nsteps_ref, off_ref, m_ref,
                 eWihT, eWhhT, ebih, ebhh,
                 dWcat, dbih, dbhh,
                 W1T, b1, W2T, b2,
                 xs_hbm, ys_hbm,
                 slab2, oslab2, h_ref, sem_in, sem_out):
    nsteps = nsteps_ref[0]

    def in_start(k, src_off):
        pltpu.make_async_copy(
            xs_hbm.at[pl.ds(pl.multiple_of(src_off, 8), BLK)],
            slab2.at[k], sem_in.at[k]).start()

    def in_wait(k):
        pltpu.make_async_copy(
            xs_hbm.at[pl.ds(0, BLK)], slab2.at[k], sem_in.at[k]).wait()

    def out_start(k, dst_off):
        pltpu.make_async_copy(
            oslab2.at[k], ys_hbm.at[pl.ds(pl.multiple_of(dst_off, 8), BLK)],
            sem_out.at[k]).start()

    def out_wait(k):
        pltpu.make_async_copy(
            oslab2.at[k], ys_hbm.at[pl.ds(0, BLK)], sem_out.at[k]).wait()

    in_start(0, off_ref[0])
    h_ref[...] = jnp.zeros_like(h_ref)

    def enc_step(t, g):
        o_t = off_ref[t]
        nblk = (off_ref[t + 1] - o_t + BLK - 1) // BLK

        def blk(b, g):
            k = lax.rem(g, 2)
            b0 = pl.multiple_of(b * BLK, BLK)
            last = b + 1 == nblk
            nxt_off = jnp.where(last, off_ref[t + 1], o_t + b0 + BLK)

            @pl.when(jnp.logical_or(jnp.logical_not(last), t + 1 < nsteps))
            def _():
                in_start(lax.rem(g + 1, 2), nxt_off)

            in_wait(k)
            h = h_ref[pl.ds(b0, BLK)]
            gi = jnp.dot(slab2[k].astype(jnp.bfloat16), eWihT[...],
                         preferred_element_type=jnp.float32) + ebih[...]
            gh = jnp.dot(h.astype(jnp.bfloat16), eWhhT[...],
                         preferred_element_type=jnp.float32) + ebhh[...]
            hn = _gru_update(gi, gh, h)
            mask = m_ref[pl.ds(b0, BLK)] > t
            h_ref[pl.ds(b0, BLK)] = jnp.where(mask, hn, h)
            return g + 1

        return lax.fori_loop(0, nblk, blk, g)

    lax.fori_loop(0, nsteps, enc_step, 0)

    def dec_step(i, g):
        o_i = off_ref[i]
        nblk = (off_ref[i + 1] - o_i + BLK - 1) // BLK

        def blk(b, g):
            k = lax.rem(g, 2)
            b0 = pl.multiple_of(b * BLK, BLK)
            hid = h_ref[pl.ds(b0, BLK)]
            hb = hid.astype(jnp.bfloat16)
            curr = jnp.where(i == 0, jnp.zeros_like(hb), hb)
            H3 = dbih.shape[-1]
            gi = jnp.dot(curr, dWcat[:, :H3],
                         preferred_element_type=jnp.float32) + dbih[...]
            gh = jnp.dot(hb, dWcat[:, H3:],
                         preferred_element_type=jnp.float32) + dbhh[...]
            hidn = _gru_update(gi, gh, hid)
            h_ref[pl.ds(b0, BLK)] = hidn
            h1 = jnp.maximum(
                jnp.dot(hidn.astype(jnp.bfloat16), W1T[...],
                        preferred_element_type=jnp.float32) + b1[...], 0.0)
            xo = (jnp.dot(h1.astype(jnp.bfloat16), W2T[...],
                          preferred_element_type=jnp.float32) + b2[...])

            @pl.when(g >= 2)
            def _():
                out_wait(k)

            oslab2[k] = xo
            out_start(k, o_i + b0)
            return g + 1

        return lax.fori_loop(0, nblk, blk, g)

    gd = lax.fori_loop(0, nsteps, dec_step, 0)

    @pl.when(gd >= 1)
    def _():
        out_wait(lax.rem(gd + 1, 2))

    @pl.when(gd >= 2)
    def _():
        out_wait(lax.rem(gd, 2))


def _tc_gru(nsteps, off, m_col, weights, xs, T, dim):
    smem = pl.BlockSpec(memory_space=pltpu.SMEM)
    vmem = pl.BlockSpec(memory_space=pltpu.VMEM)
    anyspace = pl.BlockSpec(memory_space=pl.ANY)
    return pl.pallas_call(
        _tc_gru_body,
        in_specs=[smem, smem, vmem] + [vmem] * 11 + [anyspace],
        out_specs=anyspace,
        out_shape=jax.ShapeDtypeStruct((8 * T + NB, dim), jnp.float32),
        scratch_shapes=[
            pltpu.VMEM((2, BLK, dim), jnp.float32),
            pltpu.VMEM((2, BLK, dim), jnp.float32),
            pltpu.VMEM((NB, weights[1].shape[0]), jnp.float32),
            pltpu.SemaphoreType.DMA((2,)),
            pltpu.SemaphoreType.DMA((2,)),
        ],
    )(nsteps, off, m_col, *weights, xs)


@functools.cache
def _sc_kernels(T, dim):
    info = plsc.get_sparse_core_info()
    nc, ns = info.num_cores, info.num_subcores
    nw = nc * ns
    rows_w = T // nw
    CH = 128
    nch = rows_w // CH
    mesh = plsc.VectorSubcoreMesh(core_axis_name="c", subcore_axis_name="s")
    params = pltpu.CompilerParams(needs_layout_passes=False)

    AT = 8 * T + NB

    @functools.partial(
        pl.kernel, mesh=mesh, compiler_params=params,
        out_type=[jax.ShapeDtypeStruct((AT, dim), jnp.float32),
                  jax.ShapeDtypeStruct((T // CH, CH), jnp.int32)],
        scratch_types=[
            pltpu.VMEM((rows_w,), jnp.int32),
            pltpu.VMEM((NB,), jnp.int32),
            pltpu.VMEM((NB,), jnp.int32),
            pltpu.VMEM((T,), jnp.int32),
            pltpu.VMEM((nch, CH), jnp.int32),
            pltpu.VMEM((2, CH, dim), jnp.float32),
            pltpu.SemaphoreType.DMA((2,)),
            pltpu.SemaphoreType.DMA((2,)),
        ],
    )
    def sc_scatter(x_hbm, batch_hbm, ptr_hbm, r_hbm, off_hbm,
                   xs_hbm, dest_hbm,
                   b_v, ptr_v, r_v, off_v, idx2d, rows2, seml, sems):
        wid = lax.axis_index("s") * nc + lax.axis_index("c")
        base = wid * rows_w

        def load(j, k):
            return pltpu.make_async_copy(
                x_hbm.at[pl.ds(base + j * CH, CH)], rows2.at[k], seml.at[k])

        def scat(j, k):
            return pltpu.make_async_copy(
                rows2.at[k], xs_hbm.at[idx2d.at[j]], sems.at[k])

        load(0, 0).start()
        pltpu.sync_copy(batch_hbm.at[pl.ds(base, rows_w)], b_v)
        pltpu.sync_copy(ptr_hbm, ptr_v)
        pltpu.sync_copy(r_hbm, r_v)
        pltpu.sync_copy(off_hbm, off_v)
        lanes = jnp.arange(LANES, dtype=jnp.int32)
        for j in range(nch):
            for g in range(CH // LANES):
                q = j * CH + g * LANES
                s16 = b_v[pl.ds(q, LANES)]
                ptr16 = plsc.load_gather(ptr_v, [s16])
                r16 = plsc.load_gather(r_v, [s16])
                t16 = (lanes + (base + q)) - ptr16
                off16 = plsc.load_gather(off_v, [t16])
                idx2d[j, pl.ds(g * LANES, LANES)] = off16 + r16
        for j in range(nch):
            k = j % 2
            load(j, k).wait()
            scat(j, k).start()
            if j + 1 < nch:
                if j >= 1:
                    scat(j - 1, 1 - k).wait()
                load(j + 1, 1 - k).start()
        scat(nch - 2, nch % 2).wait()
        scat(nch - 1, (nch - 1) % 2).wait()
        pltpu.sync_copy(idx2d, dest_hbm.at[pl.ds(wid * nch, nch)])

    @functools.partial(
        pl.kernel, mesh=mesh, compiler_params=params,
        out_type=jax.ShapeDtypeStruct((T, dim), jnp.float32),
        scratch_types=[
            pltpu.VMEM((nch, CH), jnp.int32),
            pltpu.VMEM((2, CH, dim), jnp.float32),
            pltpu.SemaphoreType.DMA((2,)),
            pltpu.SemaphoreType.DMA((2,)),
        ],
    )
    def sc_gather(ys_hbm, dest_hbm, out_hbm, idx2d, rows2, semg, semw):
        wid = lax.axis_index("s") * nc + lax.axis_index("c")
        base = wid * rows_w

        def gath(j, k):
            return pltpu.make_async_copy(
                ys_hbm.at[idx2d.at[j]], rows2.at[k], semg.at[k])

        def store(j, k):
            return pltpu.make_async_copy(
                rows2.at[k], out_hbm.at[pl.ds(base + j * CH, CH)], semw.at[k])

        pltpu.sync_copy(dest_hbm.at[pl.ds(wid * nch, nch)], idx2d)
        gath(0, 0).start()
        for j in range(nch):
            k = j % 2
            gath(j, k).wait()
            store(j, k).start()
            if j + 1 < nch:
                if j >= 1:
                    store(j - 1, 1 - k).wait()
                gath(j + 1, 1 - k).start()
        store(nch - 2, nch % 2).wait()
        store(nch - 1, (nch - 1) % 2).wait()

    return sc_scatter, sc_gather


def kernel(x, batch, enc_Wih, enc_Whh, enc_bih, enc_bhh,
           dec_Wih, dec_Whh, dec_bih, dec_bhh,
           map_W1, map_b1, map_W2, map_b2):
    T, dim = x.shape

    sb = batch.astype(jnp.int32)
    seg = jnp.arange(NB, dtype=jnp.int32)
    ends = jnp.sum((sb[:, None] <= seg[None, :]).astype(jnp.int32), axis=0)
    ptr = jnp.concatenate([jnp.zeros((1,), jnp.int32), ends[:-1]])
    n = ends - ptr
    order = jnp.argsort(-n, stable=True).astype(jnp.int32)
    m_desc = n[order]
    rank = jnp.argsort(order).astype(jnp.int32)
    max_n = m_desc[0]
    TCUT = 2048
    TOPK = T // TCUT
    t_lo = jnp.arange(TCUT, dtype=jnp.int32)
    c_lo = NB - jnp.sum((m_desc[None, :] <= t_lo[:, None]).astype(jnp.int32),
                        axis=1)
    t_hi = jnp.arange(TCUT, T, dtype=jnp.int32)
    c_hi = jnp.sum((m_desc[None, :TOPK] > t_hi[:, None]).astype(jnp.int32),
                   axis=1)
    c = jnp.concatenate([c_lo, c_hi])
    c8 = ((c + 7) // 8) * 8
    off = jnp.concatenate([jnp.zeros((1,), jnp.int32),
                           jnp.cumsum(c8, dtype=jnp.int32)])

    sc_scatter, sc_gather = _sc_kernels(T, dim)
    xs, dest = sc_scatter(x, sb, ptr, rank, off[:T])

    bf16 = jnp.bfloat16
    weights = (
        enc_Wih.T.astype(bf16), enc_Whh.T.astype(bf16),
        enc_bih.reshape(1, -1), enc_bhh.reshape(1, -1),
        jnp.concatenate([dec_Wih.T, dec_Whh.T], axis=1).astype(bf16),
        dec_bih.reshape(1, -1), dec_bhh.reshape(1, -1),
        map_W1.T.astype(bf16), map_b1.reshape(1, -1),
        map_W2.T.astype(bf16), map_b2.reshape(1, -1),
    )
    ys = _tc_gru(max_n.reshape(1), off, m_desc[:, None], weights, xs, T, dim)
    x_flat = sc_gather(ys, dest)
    return (x_flat, batch)
